# baseline (speedup 1.0000x reference)
import functools
import jax
import jax.numpy as jnp
from jax.experimental import pallas as pl


def _tc_body(a_ref, p_ref, out_ref, *, batch, col_chunk):
    a = a_ref[...]
    p = p_ref[...]
    a_n = a * jax.lax.rsqrt(jnp.sum(a * a, axis=1, keepdims=True))
    p_n = p * jax.lax.rsqrt(jnp.sum(p * p, axis=1, keepdims=True))

    eye = (jax.lax.broadcasted_iota(jnp.int32, (col_chunk, col_chunk), 0) ==
           jax.lax.broadcasted_iota(jnp.int32, (col_chunk, col_chunk), 1))
    chunk_mins = []
    for c in range(batch // col_chunk):
        lo = c * col_chunk
        hi = lo + col_chunk
        p_c = p_n[lo:hi, :]
        s_c = jax.lax.dot_general(a_n.astype(jnp.bfloat16),
                                  p_c.astype(jnp.bfloat16),
                                  (((1,), (1,)), ((), ())),
                                  preferred_element_type=jnp.float32)
        parts = []
        if lo > 0:
            parts.append(jnp.min(s_c[:lo, :], axis=1, keepdims=True))
        mid = jnp.where(eye, jnp.inf, s_c[lo:hi, :])
        parts.append(jnp.min(mid, axis=1, keepdims=True))
        if hi < batch:
            parts.append(jnp.min(s_c[hi:, :], axis=1, keepdims=True))
        chunk_mins.append(jnp.concatenate(parts, axis=0))
    an = chunk_mins[0]
    for m in chunk_mins[1:]:
        an = jnp.minimum(an, m)
    ap = jnp.sum(a_n * p_n, axis=1, keepdims=True)
    loss = jnp.sum(jnp.maximum(1.0 + ap - an, 0.0)) * (1.0 / batch)
    out_ref[...] = jnp.full(out_ref.shape, loss, jnp.float32)


def kernel(anchor, positive):
    batch, dim = anchor.shape
    out = pl.pallas_call(
        functools.partial(_tc_body, batch=batch, col_chunk=512),
        out_shape=jax.ShapeDtypeStruct((8, 128), jnp.float32),
    )(anchor, positive)
    return out[0, 0]


# speedup vs baseline: 1.0010x; 1.0010x over previous
import functools
import jax
import jax.numpy as jnp
from jax.experimental import pallas as pl


def _tc_body(a_ref, p_ref, out_ref, *, batch, col_chunk):
    a = a_ref[...]
    p = p_ref[...]
    a_n = a * jax.lax.rsqrt(jnp.sum(a * a, axis=1, keepdims=True))
    p_n = p * jax.lax.rsqrt(jnp.sum(p * p, axis=1, keepdims=True))

    eye = (jax.lax.broadcasted_iota(jnp.int32, (col_chunk, col_chunk), 0) ==
           jax.lax.broadcasted_iota(jnp.int32, (col_chunk, col_chunk), 1))
    chunk_mins = []
    for c in range(batch // col_chunk):
        lo = c * col_chunk
        hi = lo + col_chunk
        p_c = p_n[lo:hi, :]
        s_c = jax.lax.dot_general(a_n, p_c, (((1,), (1,)), ((), ())),
                                  preferred_element_type=jnp.float32)
        parts = []
        if lo > 0:
            parts.append(jnp.min(s_c[:lo, :], axis=1, keepdims=True))
        mid = jnp.where(eye, jnp.inf, s_c[lo:hi, :])
        parts.append(jnp.min(mid, axis=1, keepdims=True))
        if hi < batch:
            parts.append(jnp.min(s_c[hi:, :], axis=1, keepdims=True))
        chunk_mins.append(jnp.concatenate(parts, axis=0))
    an = chunk_mins[0]
    for m in chunk_mins[1:]:
        an = jnp.minimum(an, m)
    ap = jnp.sum(a_n * p_n, axis=1, keepdims=True)
    loss = jnp.sum(jnp.maximum(1.0 + ap - an, 0.0)) * (1.0 / batch)
    out_ref[...] = jnp.full(out_ref.shape, loss, jnp.float32)


def kernel(anchor, positive):
    batch, dim = anchor.shape
    out = pl.pallas_call(
        functools.partial(_tc_body, batch=batch, col_chunk=512),
        out_shape=jax.ShapeDtypeStruct((8, 128), jnp.float32),
    )(anchor, positive)
    return out[0, 0]


# speedup vs baseline: 1.0047x; 1.0036x over previous
import functools
import jax
import jax.numpy as jnp
from jax.experimental import pallas as pl


def _tc_body(a_ref, p_ref, out_ref, *, batch, col_chunk):
    a = a_ref[...]
    p = p_ref[...]
    a_n = a * jax.lax.rsqrt(jnp.sum(a * a, axis=1, keepdims=True))
    p_n = p * jax.lax.rsqrt(jnp.sum(p * p, axis=1, keepdims=True))

    eye = (jax.lax.broadcasted_iota(jnp.int32, (col_chunk, col_chunk), 0) ==
           jax.lax.broadcasted_iota(jnp.int32, (col_chunk, col_chunk), 1))
    chunk_mins = []
    for c in range(batch // col_chunk):
        lo = c * col_chunk
        hi = lo + col_chunk
        p_c = p_n[lo:hi, :]
        s_c = jax.lax.dot_general(a_n, p_c, (((1,), (1,)), ((), ())),
                                  preferred_element_type=jnp.float32)
        parts = []
        if lo > 0:
            parts.append(jnp.min(s_c[:lo, :], axis=1, keepdims=True))
        mid = jnp.where(eye, jnp.inf, s_c[lo:hi, :])
        parts.append(jnp.min(mid, axis=1, keepdims=True))
        if hi < batch:
            parts.append(jnp.min(s_c[hi:, :], axis=1, keepdims=True))
        chunk_mins.append(jnp.concatenate(parts, axis=0))
    an = chunk_mins[0]
    for m in chunk_mins[1:]:
        an = jnp.minimum(an, m)
    ap = jnp.sum(a_n * p_n, axis=1, keepdims=True)
    loss = jnp.sum(jnp.maximum(1.0 + ap - an, 0.0)) * (1.0 / batch)
    out_ref[...] = jnp.full(out_ref.shape, loss, jnp.float32)


def kernel(anchor, positive):
    batch, dim = anchor.shape
    out = pl.pallas_call(
        functools.partial(_tc_body, batch=batch, col_chunk=1024),
        out_shape=jax.ShapeDtypeStruct((8, 128), jnp.float32),
    )(anchor, positive)
    return out[0, 0]
